# Initial kernel scaffold; baseline (speedup 1.0000x reference)
import jax
import jax.numpy as jnp
from jax.experimental import pallas as pl


def kernel(x, edge_index, W1, b1, W2, b2, W3, b3):
    raise NotImplementedError("write your pallas kernel here")


# speedup vs baseline: 10.9738x; 10.9738x over previous
import functools

import jax
import jax.numpy as jnp
from jax import lax
from jax.experimental import pallas as pl
from jax.experimental.pallas import tpu as pltpu
from jax.experimental.pallas import tpu_sc as plsc

N = 10000
N_PAD = 10240
E = 320000
D = 128

NC = 2
NS = 16
C = 80
EDGES_PER_TILE = E // (NC * NS)
CHUNKS = EDGES_PER_TILE // C
ROWS_PER_TILE = N_PAD // NS
ZROWS = 80

DEGW = 128

_MESH = plsc.VectorSubcoreMesh(core_axis_name="c", subcore_axis_name="s")


def _zero_vmem(ref, rows, cols):
    z = jnp.zeros((16,), jnp.float32)

    def body(i, _):
        for j in range(cols // 16):
            ref[i, pl.ds(j * 16, 16)] = z
        return 0

    lax.fori_loop(0, rows, body, 0)


@functools.partial(
    pl.kernel,
    mesh=_MESH,
    out_type=jax.ShapeDtypeStruct((NC, N_PAD, DEGW), jnp.float32),
    scratch_types=[
        pltpu.VMEM((C,), jnp.int32),
        pltpu.VMEM((C, DEGW), jnp.float32),
        pltpu.VMEM((ZROWS, DEGW), jnp.float32),
        pltpu.VMEM_SHARED((N_PAD, DEGW), jnp.float32),
    ],
)
def _sc_degree(dst_hbm, out_hbm, dst_v, ones_v, zbuf, acc_sh):
    c = lax.axis_index("c")
    s = lax.axis_index("s")

    one = jnp.ones((16,), jnp.float32)

    def fill_ones(i, _):
        for j in range(DEGW // 16):
            ones_v[i, pl.ds(j * 16, 16)] = one
        return 0

    lax.fori_loop(0, C, fill_ones, 0)
    _zero_vmem(zbuf, ZROWS, DEGW)

    for k in range(ROWS_PER_TILE // ZROWS):
        pltpu.sync_copy(
            zbuf, acc_sh.at[pl.ds(s * ROWS_PER_TILE + k * ZROWS, ZROWS)]
        )
    plsc.subcore_barrier()

    base = (c * NS + s) * EDGES_PER_TILE

    def step(i, _):
        off = pl.multiple_of(base + i * C, 8)
        pltpu.sync_copy(dst_hbm.at[pl.ds(off, C)], dst_v)
        pltpu.sync_copy(ones_v, acc_sh.at[dst_v], add=True)
        return 0

    lax.fori_loop(0, CHUNKS, step, 0)
    plsc.subcore_barrier()
    pltpu.sync_copy(
        acc_sh.at[pl.ds(s * ROWS_PER_TILE, ROWS_PER_TILE)],
        out_hbm.at[c, pl.ds(s * ROWS_PER_TILE, ROWS_PER_TILE)],
    )


@functools.partial(
    pl.kernel,
    mesh=_MESH,
    out_type=jax.ShapeDtypeStruct((NC, N_PAD, D), jnp.float32),
    scratch_types=[
        pltpu.VMEM((C,), jnp.int32),
        pltpu.VMEM((C,), jnp.int32),
        pltpu.VMEM((C, D), jnp.float32),
        pltpu.VMEM((ZROWS, D), jnp.float32),
        pltpu.VMEM_SHARED((N_PAD, D), jnp.float32),
        pltpu.SemaphoreType.DMA,
    ],
)
def _sc_scatter(g_hbm, src_hbm, dst_hbm, out_hbm, src_v, dst_v, rows_v, zbuf, acc_sh, sem):
    c = lax.axis_index("c")
    s = lax.axis_index("s")

    _zero_vmem(zbuf, ZROWS, D)

    for k in range(ROWS_PER_TILE // ZROWS):
        pltpu.sync_copy(
            zbuf, acc_sh.at[pl.ds(s * ROWS_PER_TILE + k * ZROWS, ZROWS)]
        )
    plsc.subcore_barrier()

    base = (c * NS + s) * EDGES_PER_TILE

    def step(i, _):
        off = pl.multiple_of(base + i * C, 8)
        pltpu.sync_copy(src_hbm.at[pl.ds(off, C)], src_v)
        pltpu.sync_copy(dst_hbm.at[pl.ds(off, C)], dst_v)
        pltpu.async_copy(g_hbm.at[src_v], rows_v, sem).wait()
        pltpu.sync_copy(rows_v, acc_sh.at[dst_v], add=True)
        return 0

    lax.fori_loop(0, CHUNKS, step, 0)
    plsc.subcore_barrier()
    pltpu.sync_copy(
        acc_sh.at[pl.ds(s * ROWS_PER_TILE, ROWS_PER_TILE)],
        out_hbm.at[c, pl.ds(s * ROWS_PER_TILE, ROWS_PER_TILE)],
    )


_R = 2000
_GRID = N // _R


def _dinv_block(degp_ref):
    deg = degp_ref[0, :, 0] + degp_ref[1, :, 0] + 1.0
    return lax.rsqrt(jnp.maximum(deg, 1.0))


def _tc_first_body(degp_ref, x_ref, w_ref, g_ref):
    dinv = _dinv_block(degp_ref)
    h = jnp.dot(x_ref[...], w_ref[...], preferred_element_type=jnp.float32)
    g_ref[...] = h * dinv[:, None]


def _tc_mid_body(sp_ref, g_ref, degp_ref, b_ref, w_ref, gn_ref):
    dinv = _dinv_block(degp_ref)
    pre = (sp_ref[0] + sp_ref[1] + g_ref[...]) * dinv[:, None] + b_ref[...]
    a = jnp.maximum(pre, 0.0)
    gn_ref[...] = (
        jnp.dot(a, w_ref[...], preferred_element_type=jnp.float32) * dinv[:, None]
    )


def _tc_last_body(sp_ref, g_ref, degp_ref, b_ref, out_ref):
    dinv = _dinv_block(degp_ref)
    out_ref[...] = (sp_ref[0] + sp_ref[1] + g_ref[...]) * dinv[:, None] + b_ref[...]


_degp_spec = pl.BlockSpec((NC, _R, DEGW), lambda i: (0, i, 0))
_row_spec = pl.BlockSpec((_R, D), lambda i: (i, 0))
_sp_spec = pl.BlockSpec((NC, _R, D), lambda i: (0, i, 0))
_w_spec = pl.BlockSpec((D, D), lambda i: (0, 0))
_b_spec = pl.BlockSpec((1, D), lambda i: (0, 0))


def _tc_first(degp, x, w):
    return pl.pallas_call(
        _tc_first_body,
        grid=(_GRID,),
        in_specs=[_degp_spec, _row_spec, _w_spec],
        out_specs=_row_spec,
        out_shape=jax.ShapeDtypeStruct((N, D), jnp.float32),
    )(degp, x, w)


def _tc_mid(sp, g, degp, b, w):
    return pl.pallas_call(
        _tc_mid_body,
        grid=(_GRID,),
        in_specs=[_sp_spec, _row_spec, _degp_spec, _b_spec, _w_spec],
        out_specs=_row_spec,
        out_shape=jax.ShapeDtypeStruct((N, D), jnp.float32),
    )(sp, g, degp, b, w)


def _tc_last(sp, g, degp, b):
    return pl.pallas_call(
        _tc_last_body,
        grid=(_GRID,),
        in_specs=[_sp_spec, _row_spec, _degp_spec, _b_spec],
        out_specs=_row_spec,
        out_shape=jax.ShapeDtypeStruct((N, D), jnp.float32),
    )(sp, g, degp, b)


def kernel(x, edge_index, W1, b1, W2, b2, W3, b3):
    edge_index = edge_index.astype(jnp.int32)
    src = edge_index[0]
    dst = edge_index[1]
    b1 = b1.reshape(1, D)
    b2 = b2.reshape(1, D)
    b3 = b3.reshape(1, D)

    degp = _sc_degree(dst)
    g1 = _tc_first(degp, x, W1)
    s1 = _sc_scatter(g1, src, dst)
    g2 = _tc_mid(s1, g1, degp, b1, W2)
    s2 = _sc_scatter(g2, src, dst)
    g3 = _tc_mid(s2, g2, degp, b2, W3)
    s3 = _sc_scatter(g3, src, dst)
    return _tc_last(s3, g3, degp, b3)


# speedup vs baseline: 17.5957x; 1.6034x over previous
import functools

import jax
import jax.numpy as jnp
from jax import lax
from jax.experimental import pallas as pl
from jax.experimental.pallas import tpu as pltpu
from jax.experimental.pallas import tpu_sc as plsc

N = 10000
N_PAD = 10240
E = 320000
D = 128

NC = 2
NS = 16
C = 80
EDGES_PER_TILE = E // (NC * NS)
CHUNKS = EDGES_PER_TILE // C
ROWS_PER_TILE = N_PAD // NS
ZROWS = 80

DEGW = 128

_MESH = plsc.VectorSubcoreMesh(core_axis_name="c", subcore_axis_name="s")


def _zero_vmem(ref, rows, cols):
    z = jnp.zeros((16,), jnp.float32)

    def body(i, _):
        for j in range(cols // 16):
            ref[i, pl.ds(j * 16, 16)] = z
        return 0

    lax.fori_loop(0, rows, body, 0)


@functools.partial(
    pl.kernel,
    mesh=_MESH,
    out_type=jax.ShapeDtypeStruct((NC, N_PAD, DEGW), jnp.float32),
    scratch_types=[
        pltpu.VMEM((C,), jnp.int32),
        pltpu.VMEM((C,), jnp.int32),
        pltpu.VMEM((C, DEGW), jnp.float32),
        pltpu.VMEM((ZROWS, DEGW), jnp.float32),
        pltpu.VMEM_SHARED((N_PAD, DEGW), jnp.float32),
        pltpu.SemaphoreType.DMA,
        pltpu.SemaphoreType.DMA,
    ],
)
def _sc_degree(dst_hbm, out_hbm, dst0_v, dst1_v, ones_v, zbuf, acc_sh, sem0, sem1):
    c = lax.axis_index("c")
    s = lax.axis_index("s")
    sems = (sem0, sem1)
    dst_v = (dst0_v, dst1_v)

    one = jnp.ones((16,), jnp.float32)

    def fill_ones(i, _):
        for j in range(DEGW // 16):
            ones_v[i, pl.ds(j * 16, 16)] = one
        return 0

    lax.fori_loop(0, C, fill_ones, 0)
    _zero_vmem(zbuf, ZROWS, DEGW)

    for k in range(ROWS_PER_TILE // ZROWS):
        pltpu.sync_copy(
            zbuf, acc_sh.at[pl.ds(s * ROWS_PER_TILE + k * ZROWS, ZROWS)]
        )
    plsc.subcore_barrier()

    base = (c * NS + s) * EDGES_PER_TILE

    def off(i):
        return pl.multiple_of(base + i * C, 8)

    pltpu.sync_copy(dst_hbm.at[pl.ds(off(0), C)], dst_v[0])
    pending = [None, None]
    for i in range(CHUNKS):
        b = i % 2
        if i + 1 < CHUNKS:
            pending[1 - b] = pltpu.async_copy(
                dst_hbm.at[pl.ds(off(i + 1), C)], dst_v[1 - b], sems[1 - b]
            )
        pltpu.sync_copy(ones_v, acc_sh.at[dst_v[b]], add=True)
        if i + 1 < CHUNKS:
            pending[1 - b].wait()
    plsc.subcore_barrier()
    pltpu.sync_copy(
        acc_sh.at[pl.ds(s * ROWS_PER_TILE, ROWS_PER_TILE)],
        out_hbm.at[c, pl.ds(s * ROWS_PER_TILE, ROWS_PER_TILE)],
    )


@functools.partial(
    pl.kernel,
    mesh=_MESH,
    out_type=jax.ShapeDtypeStruct((NC, N_PAD, D), jnp.float32),
    scratch_types=[
        pltpu.VMEM((C,), jnp.int32),
        pltpu.VMEM((C,), jnp.int32),
        pltpu.VMEM((C,), jnp.int32),
        pltpu.VMEM((C,), jnp.int32),
        pltpu.VMEM((2, C, D), jnp.float32),
        pltpu.VMEM((ZROWS, D), jnp.float32),
        pltpu.VMEM_SHARED((N_PAD, D), jnp.float32),
        pltpu.SemaphoreType.DMA,
        pltpu.SemaphoreType.DMA,
    ],
)
def _sc_scatter(g_hbm, src_hbm, dst_hbm, out_hbm, src0_v, src1_v, dst0_v, dst1_v, rows_v, zbuf, acc_sh, sem0, sem1):
    c = lax.axis_index("c")
    s = lax.axis_index("s")
    sems = (sem0, sem1)
    src_v = (src0_v, src1_v)
    dst_v = (dst0_v, dst1_v)

    _zero_vmem(zbuf, ZROWS, D)

    for k in range(ROWS_PER_TILE // ZROWS):
        pltpu.sync_copy(
            zbuf, acc_sh.at[pl.ds(s * ROWS_PER_TILE + k * ZROWS, ZROWS)]
        )
    plsc.subcore_barrier()

    base = (c * NS + s) * EDGES_PER_TILE

    def idx_load(i, b):
        off = pl.multiple_of(base + i * C, 8)
        pltpu.sync_copy(src_hbm.at[pl.ds(off, C)], src_v[b])
        pltpu.sync_copy(dst_hbm.at[pl.ds(off, C)], dst_v[b])

    gat = [None, None]
    idx_load(0, 0)
    gat[0] = pltpu.async_copy(g_hbm.at[src_v[0]], rows_v.at[0], sems[0])
    idx_load(1, 1)
    for i in range(CHUNKS):
        b = i % 2
        if i + 1 < CHUNKS:
            gat[1 - b] = pltpu.async_copy(
                g_hbm.at[src_v[1 - b]], rows_v.at[1 - b], sems[1 - b]
            )
        gat[b].wait()
        pltpu.sync_copy(rows_v.at[b], acc_sh.at[dst_v[b]], add=True)
        if i + 2 < CHUNKS:
            idx_load(i + 2, b)
    plsc.subcore_barrier()
    pltpu.sync_copy(
        acc_sh.at[pl.ds(s * ROWS_PER_TILE, ROWS_PER_TILE)],
        out_hbm.at[c, pl.ds(s * ROWS_PER_TILE, ROWS_PER_TILE)],
    )


_R = 2000
_GRID = N // _R


def _dinv_block(degp_ref):
    deg = degp_ref[0, :, 0] + degp_ref[1, :, 0] + 1.0
    return lax.rsqrt(jnp.maximum(deg, 1.0))


def _tc_first_body(degp_ref, x_ref, w_ref, g_ref):
    dinv = _dinv_block(degp_ref)
    h = jnp.dot(x_ref[...], w_ref[...], preferred_element_type=jnp.float32)
    g_ref[...] = h * dinv[:, None]


def _tc_mid_body(sp_ref, g_ref, degp_ref, b_ref, w_ref, gn_ref):
    dinv = _dinv_block(degp_ref)
    pre = (sp_ref[0] + sp_ref[1] + g_ref[...]) * dinv[:, None] + b_ref[...]
    a = jnp.maximum(pre, 0.0)
    gn_ref[...] = (
        jnp.dot(a, w_ref[...], preferred_element_type=jnp.float32) * dinv[:, None]
    )


def _tc_last_body(sp_ref, g_ref, degp_ref, b_ref, out_ref):
    dinv = _dinv_block(degp_ref)
    out_ref[...] = (sp_ref[0] + sp_ref[1] + g_ref[...]) * dinv[:, None] + b_ref[...]


_degp_spec = pl.BlockSpec((NC, _R, DEGW), lambda i: (0, i, 0))
_row_spec = pl.BlockSpec((_R, D), lambda i: (i, 0))
_sp_spec = pl.BlockSpec((NC, _R, D), lambda i: (0, i, 0))
_w_spec = pl.BlockSpec((D, D), lambda i: (0, 0))
_b_spec = pl.BlockSpec((1, D), lambda i: (0, 0))


def _tc_first(degp, x, w):
    return pl.pallas_call(
        _tc_first_body,
        grid=(_GRID,),
        in_specs=[_degp_spec, _row_spec, _w_spec],
        out_specs=_row_spec,
        out_shape=jax.ShapeDtypeStruct((N, D), jnp.float32),
    )(degp, x, w)


def _tc_mid(sp, g, degp, b, w):
    return pl.pallas_call(
        _tc_mid_body,
        grid=(_GRID,),
        in_specs=[_sp_spec, _row_spec, _degp_spec, _b_spec, _w_spec],
        out_specs=_row_spec,
        out_shape=jax.ShapeDtypeStruct((N, D), jnp.float32),
    )(sp, g, degp, b, w)


def _tc_last(sp, g, degp, b):
    return pl.pallas_call(
        _tc_last_body,
        grid=(_GRID,),
        in_specs=[_sp_spec, _row_spec, _degp_spec, _b_spec],
        out_specs=_row_spec,
        out_shape=jax.ShapeDtypeStruct((N, D), jnp.float32),
    )(sp, g, degp, b)


def kernel(x, edge_index, W1, b1, W2, b2, W3, b3):
    edge_index = edge_index.astype(jnp.int32)
    src = edge_index[0]
    dst = edge_index[1]
    b1 = b1.reshape(1, D)
    b2 = b2.reshape(1, D)
    b3 = b3.reshape(1, D)

    degp = _sc_degree(dst)
    g1 = _tc_first(degp, x, W1)
    s1 = _sc_scatter(g1, src, dst)
    g2 = _tc_mid(s1, g1, degp, b1, W2)
    s2 = _sc_scatter(g2, src, dst)
    g3 = _tc_mid(s2, g2, degp, b2, W3)
    s3 = _sc_scatter(g3, src, dst)
    return _tc_last(s3, g3, degp, b3)


# speedup vs baseline: 20.7885x; 1.1814x over previous
import functools

import jax
import jax.numpy as jnp
from jax import lax
from jax.experimental import pallas as pl
from jax.experimental.pallas import tpu as pltpu
from jax.experimental.pallas import tpu_sc as plsc

N = 10000
N_PAD = 10240
E = 320000
D = 128

NC = 2
NS = 16
C = 80
EDGES_PER_TILE = E // (NC * NS)
CHUNKS = EDGES_PER_TILE // C
ROWS_PER_TILE = N_PAD // NS
ZROWS = 80

CF = 80
CHUNKS_F = EDGES_PER_TILE // CF
NBUF = 3

DEGW = 128

_MESH = plsc.VectorSubcoreMesh(core_axis_name="c", subcore_axis_name="s")


def _zero_vmem(ref, rows, cols):
    z = jnp.zeros((16,), jnp.float32)

    def body(i, _):
        for j in range(cols // 16):
            ref[i, pl.ds(j * 16, 16)] = z
        return 0

    lax.fori_loop(0, rows, body, 0)


@functools.partial(
    pl.kernel,
    mesh=_MESH,
    out_type=jax.ShapeDtypeStruct((NC, N_PAD, DEGW), jnp.float32),
    scratch_types=[
        pltpu.VMEM((C,), jnp.int32),
        pltpu.VMEM((C,), jnp.int32),
        pltpu.VMEM((C, DEGW), jnp.float32),
        pltpu.VMEM((ZROWS, DEGW), jnp.float32),
        pltpu.VMEM_SHARED((N_PAD, DEGW), jnp.float32),
        pltpu.SemaphoreType.DMA,
        pltpu.SemaphoreType.DMA,
    ],
)
def _sc_degree(dst_hbm, out_hbm, dst0_v, dst1_v, ones_v, zbuf, acc_sh, sem0, sem1):
    c = lax.axis_index("c")
    s = lax.axis_index("s")
    sems = (sem0, sem1)
    dst_v = (dst0_v, dst1_v)

    one = jnp.ones((16,), jnp.float32)

    def fill_ones(i, _):
        for j in range(DEGW // 16):
            ones_v[i, pl.ds(j * 16, 16)] = one
        return 0

    lax.fori_loop(0, C, fill_ones, 0)
    _zero_vmem(zbuf, ZROWS, DEGW)

    for k in range(ROWS_PER_TILE // ZROWS):
        pltpu.sync_copy(
            zbuf, acc_sh.at[pl.ds(s * ROWS_PER_TILE + k * ZROWS, ZROWS)]
        )
    plsc.subcore_barrier()

    base = (c * NS + s) * EDGES_PER_TILE

    def off(i):
        return pl.multiple_of(base + i * C, 8)

    pltpu.sync_copy(dst_hbm.at[pl.ds(off(0), C)], dst_v[0])
    pending = [None, None]
    for i in range(CHUNKS):
        b = i % 2
        if i + 1 < CHUNKS:
            pending[1 - b] = pltpu.async_copy(
                dst_hbm.at[pl.ds(off(i + 1), C)], dst_v[1 - b], sems[1 - b]
            )
        pltpu.sync_copy(ones_v, acc_sh.at[dst_v[b]], add=True)
        if i + 1 < CHUNKS:
            pending[1 - b].wait()
    plsc.subcore_barrier()
    pltpu.sync_copy(
        acc_sh.at[pl.ds(s * ROWS_PER_TILE, ROWS_PER_TILE)],
        out_hbm.at[c, pl.ds(s * ROWS_PER_TILE, ROWS_PER_TILE)],
    )


@functools.partial(
    pl.kernel,
    mesh=_MESH,
    out_type=jax.ShapeDtypeStruct((NC, N_PAD, D), jnp.float32),
    scratch_types=(
        [pltpu.VMEM((CF,), jnp.int32) for _ in range(2 * NBUF)]
        + [
            pltpu.VMEM((NBUF, CF, D), jnp.float32),
            pltpu.VMEM((ZROWS, D), jnp.float32),
            pltpu.VMEM_SHARED((N_PAD, D), jnp.float32),
        ]
        + [pltpu.SemaphoreType.DMA for _ in range(2 * NBUF)]
    ),
)
def _sc_scatter(g_hbm, src_hbm, dst_hbm, out_hbm, *scr):
    src_v = scr[0:NBUF]
    dst_v = scr[NBUF : 2 * NBUF]
    rows_v = scr[2 * NBUF]
    zbuf = scr[2 * NBUF + 1]
    acc_sh = scr[2 * NBUF + 2]
    sem_g = scr[2 * NBUF + 3 : 3 * NBUF + 3]
    sem_s = scr[3 * NBUF + 3 : 4 * NBUF + 3]

    c = lax.axis_index("c")
    s = lax.axis_index("s")

    _zero_vmem(zbuf, ZROWS, D)

    for k in range(ROWS_PER_TILE // ZROWS):
        pltpu.sync_copy(
            zbuf, acc_sh.at[pl.ds(s * ROWS_PER_TILE + k * ZROWS, ZROWS)]
        )
    plsc.subcore_barrier()

    base = (c * NS + s) * EDGES_PER_TILE
    g_c = g_hbm
    out_c = out_hbm.at[c]

    def idx_load(i, b):
        off = pl.multiple_of(base + i * CF, 8)
        pltpu.sync_copy(src_hbm.at[pl.ds(off, CF)], src_v[b])
        pltpu.sync_copy(dst_hbm.at[pl.ds(off, CF)], dst_v[b])

    gat = [None] * NBUF
    sca = [None] * NBUF
    for j in range(min(2, CHUNKS_F)):
        idx_load(j, j)
        gat[j] = pltpu.async_copy(g_c.at[src_v[j]], rows_v.at[j], sem_g[j])
    for i in range(CHUNKS_F):
        b = i % NBUF
        gat[b].wait()
        sca[b] = pltpu.async_copy(
            rows_v.at[b], acc_sh.at[dst_v[b]], sem_s[b], add=True
        )
        nxt = i + 2
        if nxt < CHUNKS_F:
            nb = nxt % NBUF
            if sca[nb] is not None:
                sca[nb].wait()
                sca[nb] = None
            idx_load(nxt, nb)
            gat[nb] = pltpu.async_copy(
                g_c.at[src_v[nb]], rows_v.at[nb], sem_g[nb]
            )
    for b in range(NBUF):
        if sca[b] is not None:
            sca[b].wait()
    plsc.subcore_barrier()
    pltpu.sync_copy(
        acc_sh.at[pl.ds(s * ROWS_PER_TILE, ROWS_PER_TILE)],
        out_c.at[pl.ds(s * ROWS_PER_TILE, ROWS_PER_TILE)],
    )


_R = 2000
_GRID = N // _R


def _dinv_block(degp_ref):
    deg = degp_ref[0, :, 0] + degp_ref[1, :, 0] + 1.0
    return lax.rsqrt(jnp.maximum(deg, 1.0))


def _tc_first_body(degp_ref, x_ref, w_ref, g_ref):
    dinv = _dinv_block(degp_ref)
    h = jnp.dot(x_ref[...], w_ref[...], preferred_element_type=jnp.float32)
    g_ref[...] = h * dinv[:, None]


def _tc_mid_body(sp_ref, g_ref, degp_ref, b_ref, w_ref, gn_ref):
    dinv = _dinv_block(degp_ref)
    pre = (sp_ref[0] + sp_ref[1] + g_ref[...]) * dinv[:, None] + b_ref[...]
    a = jnp.maximum(pre, 0.0)
    gn_ref[...] = (
        jnp.dot(a, w_ref[...], preferred_element_type=jnp.float32) * dinv[:, None]
    )


def _tc_last_body(sp_ref, g_ref, degp_ref, b_ref, out_ref):
    dinv = _dinv_block(degp_ref)
    out_ref[...] = (sp_ref[0] + sp_ref[1] + g_ref[...]) * dinv[:, None] + b_ref[...]


_degp_spec = pl.BlockSpec((NC, _R, DEGW), lambda i: (0, i, 0))
_row_spec = pl.BlockSpec((_R, D), lambda i: (i, 0))
_sp_spec = pl.BlockSpec((NC, _R, D), lambda i: (0, i, 0))
_w_spec = pl.BlockSpec((D, D), lambda i: (0, 0))
_b_spec = pl.BlockSpec((1, D), lambda i: (0, 0))


def _tc_first(degp, x, w):
    return pl.pallas_call(
        _tc_first_body,
        grid=(_GRID,),
        in_specs=[_degp_spec, _row_spec, _w_spec],
        out_specs=_row_spec,
        out_shape=jax.ShapeDtypeStruct((N, D), jnp.float32),
    )(degp, x, w)


def _tc_mid(sp, g, degp, b, w):
    return pl.pallas_call(
        _tc_mid_body,
        grid=(_GRID,),
        in_specs=[_sp_spec, _row_spec, _degp_spec, _b_spec, _w_spec],
        out_specs=_row_spec,
        out_shape=jax.ShapeDtypeStruct((N, D), jnp.float32),
    )(sp, g, degp, b, w)


def _tc_last(sp, g, degp, b):
    return pl.pallas_call(
        _tc_last_body,
        grid=(_GRID,),
        in_specs=[_sp_spec, _row_spec, _degp_spec, _b_spec],
        out_specs=_row_spec,
        out_shape=jax.ShapeDtypeStruct((N, D), jnp.float32),
    )(sp, g, degp, b)


def kernel(x, edge_index, W1, b1, W2, b2, W3, b3):
    edge_index = edge_index.astype(jnp.int32)
    src = edge_index[0]
    dst = edge_index[1]
    b1 = b1.reshape(1, D)
    b2 = b2.reshape(1, D)
    b3 = b3.reshape(1, D)

    degp = _sc_degree(dst)
    g1 = _tc_first(degp, x, W1)
    s1 = _sc_scatter(g1, src, dst)
    g2 = _tc_mid(s1, g1, degp, b1, W2)
    s2 = _sc_scatter(g2, src, dst)
    g3 = _tc_mid(s2, g2, degp, b2, W3)
    s3 = _sc_scatter(g3, src, dst)
    return _tc_last(s3, g3, degp, b3)


# speedup vs baseline: 32.3591x; 1.5566x over previous
import functools

import jax
import jax.numpy as jnp
from jax import lax
from jax.experimental import pallas as pl
from jax.experimental.pallas import tpu as pltpu
from jax.experimental.pallas import tpu_sc as plsc

N = 10000
N_PAD = 10240
E = 320000
D = 128

NC = 2
NS = 16
EDGES_PER_TILE = E // (NC * NS)
ROWS_PER_TILE = N_PAD // NS
ZROWS = 80

CF = 80
CHUNKS_F = EDGES_PER_TILE // CF
NBUF = 3
NDST = 4

_MESH = plsc.VectorSubcoreMesh(core_axis_name="c", subcore_axis_name="s")


def _zero_vmem(ref, rows, cols):
    z = jnp.zeros((16,), jnp.float32)

    def body(i, _):
        for j in range(cols // 16):
            ref[i, pl.ds(j * 16, 16)] = z
        return 0

    lax.fori_loop(0, rows, body, 0)


@functools.partial(
    pl.kernel,
    mesh=_MESH,
    out_type=jax.ShapeDtypeStruct((NC * NS, N_PAD), jnp.float32),
    scratch_types=[
        pltpu.VMEM((EDGES_PER_TILE,), jnp.int32),
        pltpu.VMEM((N_PAD,), jnp.float32),
    ],
    compiler_params=pltpu.CompilerParams(needs_layout_passes=False),
)
def _sc_degree(dst_hbm, out_hbm, idx_all, deg_local):
    c = lax.axis_index("c")
    s = lax.axis_index("s")
    w = c * NS + s

    z = jnp.zeros((16,), jnp.float32)

    def zbody(i, _):
        deg_local[pl.ds(i * 16, 16)] = z
        return 0

    lax.fori_loop(0, N_PAD // 16, zbody, 0)

    base = pl.multiple_of(w * EDGES_PER_TILE, 8)
    pltpu.sync_copy(dst_hbm.at[pl.ds(base, EDGES_PER_TILE)], idx_all)

    one = jnp.ones((16,), jnp.float32)

    def body(k, _):
        idx_v = idx_all[pl.ds(k * 16, 16)]
        plsc.addupdate_scatter(deg_local, [idx_v], one)
        return 0

    lax.fori_loop(0, EDGES_PER_TILE // 16, body, 0)

    pltpu.sync_copy(deg_local, out_hbm.at[w])


@functools.partial(
    pl.kernel,
    mesh=_MESH,
    out_type=jax.ShapeDtypeStruct((NC, N_PAD, D), jnp.float32),
    scratch_types=(
        [pltpu.VMEM((CF,), jnp.int32) for _ in range(NDST)]
        + [
            pltpu.VMEM((EDGES_PER_TILE,), jnp.int32),
            pltpu.VMEM((NBUF, CF, D), jnp.float32),
            pltpu.VMEM_SHARED((N_PAD, D), jnp.float32),
        ]
        + [pltpu.SemaphoreType.DMA for _ in range(2 * NBUF + NDST)]
    ),
)
def _sc_scatter(g_hbm, src_hbm, dst_hbm, out_hbm, *scr):
    dst_v = scr[0:NDST]
    src_all = scr[NDST]
    rows_v = scr[NDST + 1]
    acc_sh = scr[NDST + 2]
    sem_g = scr[NDST + 3 : NDST + 3 + NBUF]
    sem_s = scr[NDST + 3 + NBUF : NDST + 3 + 2 * NBUF]
    sem_d = scr[NDST + 3 + 2 * NBUF :]

    c = lax.axis_index("c")
    s = lax.axis_index("s")

    base = pl.multiple_of((c * NS + s) * EDGES_PER_TILE, 8)
    out_c = out_hbm.at[c]

    _zero_vmem(rows_v.at[0], CF, D)
    for k in range(ROWS_PER_TILE // ZROWS):
        pltpu.sync_copy(
            rows_v.at[0], acc_sh.at[pl.ds(s * ROWS_PER_TILE + k * ZROWS, ZROWS)]
        )

    pltpu.sync_copy(src_hbm.at[pl.ds(base, EDGES_PER_TILE)], src_all)

    def dst_load(i, db):
        off = pl.multiple_of(base + i * CF, 8)
        return pltpu.async_copy(dst_hbm.at[pl.ds(off, CF)], dst_v[db], sem_d[db])

    def gather(i, b):
        return pltpu.async_copy(
            g_hbm.at[src_all.at[pl.ds(i * CF, CF)]], rows_v.at[b], sem_g[b]
        )

    gat = [None] * NBUF
    sca = [None] * NBUF
    dld = [None] * NDST
    for j in range(3):
        dld[j] = dst_load(j, j)
    for j in range(2):
        gat[j] = gather(j, j)
    plsc.subcore_barrier()
    for i in range(CHUNKS_F):
        b = i % NBUF
        db = i % NDST
        gat[b].wait()
        dld[db].wait()
        sca[b] = pltpu.async_copy(
            rows_v.at[b], acc_sh.at[dst_v[db]], sem_s[b], add=True
        )
        nxt = i + 2
        if nxt < CHUNKS_F:
            nb = nxt % NBUF
            if sca[nb] is not None:
                sca[nb].wait()
                sca[nb] = None
            gat[nb] = gather(nxt, nb)
        nd = i + 3
        if nd < CHUNKS_F:
            dld[nd % NDST] = dst_load(nd, nd % NDST)
    for b in range(NBUF):
        if sca[b] is not None:
            sca[b].wait()
    plsc.subcore_barrier()
    pltpu.sync_copy(
        acc_sh.at[pl.ds(s * ROWS_PER_TILE, ROWS_PER_TILE)],
        out_c.at[pl.ds(s * ROWS_PER_TILE, ROWS_PER_TILE)],
    )


_R = 2560
_GRID = N_PAD // _R


def _dinv_block(degp_ref):
    deg = jnp.sum(degp_ref[...], axis=0) + 1.0
    return lax.rsqrt(jnp.maximum(deg, 1.0))


def _tc_first_body(degp_ref, x_ref, w_ref, g_ref):
    dinv = _dinv_block(degp_ref)
    h = jnp.dot(x_ref[...], w_ref[...], preferred_element_type=jnp.float32)
    g_ref[...] = h * dinv[:, None]


def _tc_mid_body(sp_ref, g_ref, degp_ref, b_ref, w_ref, gn_ref):
    dinv = _dinv_block(degp_ref)
    pre = (sp_ref[0] + sp_ref[1] + g_ref[...]) * dinv[:, None] + b_ref[...]
    a = jnp.maximum(pre, 0.0)
    gn_ref[...] = (
        jnp.dot(a, w_ref[...], preferred_element_type=jnp.float32) * dinv[:, None]
    )


def _tc_last_body(sp_ref, g_ref, degp_ref, b_ref, out_ref):
    dinv = _dinv_block(degp_ref)
    out_ref[...] = (sp_ref[0] + sp_ref[1] + g_ref[...]) * dinv[:, None] + b_ref[...]


_degp_spec = pl.BlockSpec((NC * NS, _R), lambda i: (0, i))
_row_spec = pl.BlockSpec((_R, D), lambda i: (i, 0))
_sp_spec = pl.BlockSpec((NC, _R, D), lambda i: (0, i, 0))
_w_spec = pl.BlockSpec((D, D), lambda i: (0, 0))
_b_spec = pl.BlockSpec((1, D), lambda i: (0, 0))


def _tc_first(degp, x, w):
    return pl.pallas_call(
        _tc_first_body,
        grid=(_GRID,),
        in_specs=[_degp_spec, _row_spec, _w_spec],
        out_specs=_row_spec,
        out_shape=jax.ShapeDtypeStruct((N_PAD, D), jnp.float32),
    )(degp, x, w)


def _tc_mid(sp, g, degp, b, w):
    return pl.pallas_call(
        _tc_mid_body,
        grid=(_GRID,),
        in_specs=[_sp_spec, _row_spec, _degp_spec, _b_spec, _w_spec],
        out_specs=_row_spec,
        out_shape=jax.ShapeDtypeStruct((N_PAD, D), jnp.float32),
    )(sp, g, degp, b, w)


def _tc_last(sp, g, degp, b):
    return pl.pallas_call(
        _tc_last_body,
        grid=(_GRID,),
        in_specs=[_sp_spec, _row_spec, _degp_spec, _b_spec],
        out_specs=_row_spec,
        out_shape=jax.ShapeDtypeStruct((N_PAD, D), jnp.float32),
    )(sp, g, degp, b)


def kernel(x, edge_index, W1, b1, W2, b2, W3, b3):
    edge_index = edge_index.astype(jnp.int32)
    src = edge_index[0]
    dst = edge_index[1]
    b1 = b1.reshape(1, D)
    b2 = b2.reshape(1, D)
    b3 = b3.reshape(1, D)

    x_p = jnp.pad(x, ((0, N_PAD - N), (0, 0)))
    degp = _sc_degree(dst)
    g1 = _tc_first(degp, x_p, W1)
    s1 = _sc_scatter(g1, src, dst)
    g2 = _tc_mid(s1, g1, degp, b1, W2)
    s2 = _sc_scatter(g2, src, dst)
    g3 = _tc_mid(s2, g2, degp, b2, W3)
    s3 = _sc_scatter(g3, src, dst)
    return _tc_last(s3, g3, degp, b3)[:N]


# speedup vs baseline: 32.9487x; 1.0182x over previous
import functools

import jax
import jax.numpy as jnp
from jax import lax
from jax.experimental import pallas as pl
from jax.experimental.pallas import tpu as pltpu
from jax.experimental.pallas import tpu_sc as plsc

N = 10000
N_PAD = 10240
E = 320000
D = 128

NC = 2
NS = 16
EDGES_PER_TILE = E // (NC * NS)
ROWS_PER_TILE = N_PAD // NS
ZROWS = 80

CF = 96
CHUNKS_F = EDGES_PER_TILE // CF
CTAIL = EDGES_PER_TILE - CHUNKS_F * CF
NBUF = 3
NDST = 4

_MESH = plsc.VectorSubcoreMesh(core_axis_name="c", subcore_axis_name="s")


def _zero_vmem(ref, rows, cols):
    z = jnp.zeros((16,), jnp.float32)

    def body(i, _):
        for j in range(cols // 16):
            ref[i, pl.ds(j * 16, 16)] = z
        return 0

    lax.fori_loop(0, rows, body, 0)


@functools.partial(
    pl.kernel,
    mesh=_MESH,
    out_type=jax.ShapeDtypeStruct((NC * NS, N_PAD), jnp.float32),
    scratch_types=[
        pltpu.VMEM((EDGES_PER_TILE,), jnp.int32),
        pltpu.VMEM((N_PAD,), jnp.float32),
    ],
    compiler_params=pltpu.CompilerParams(needs_layout_passes=False),
)
def _sc_degree(dst_hbm, out_hbm, idx_all, deg_local):
    c = lax.axis_index("c")
    s = lax.axis_index("s")
    w = c * NS + s

    z = jnp.zeros((16,), jnp.float32)

    def zbody(i, _):
        deg_local[pl.ds(i * 16, 16)] = z
        return 0

    lax.fori_loop(0, N_PAD // 16, zbody, 0)

    base = pl.multiple_of(w * EDGES_PER_TILE, 8)
    pltpu.sync_copy(dst_hbm.at[pl.ds(base, EDGES_PER_TILE)], idx_all)

    one = jnp.ones((16,), jnp.float32)

    def body(k, _):
        idx_v = idx_all[pl.ds(k * 16, 16)]
        plsc.addupdate_scatter(deg_local, [idx_v], one)
        return 0

    lax.fori_loop(0, EDGES_PER_TILE // 16, body, 0)

    pltpu.sync_copy(deg_local, out_hbm.at[w])


@functools.partial(
    pl.kernel,
    mesh=_MESH,
    out_type=jax.ShapeDtypeStruct((NC, N_PAD, D), jnp.float32),
    scratch_types=(
        [pltpu.VMEM((CF,), jnp.int32) for _ in range(NDST)]
        + [
            pltpu.VMEM((CTAIL,), jnp.int32),
            pltpu.VMEM((EDGES_PER_TILE,), jnp.int32),
            pltpu.VMEM((NBUF, CF, D), jnp.float32),
            pltpu.VMEM_SHARED((N_PAD, D), jnp.float32),
        ]
        + [pltpu.SemaphoreType.DMA for _ in range(2 * NBUF + NDST)]
    ),
)
def _sc_scatter(g_hbm, src_hbm, dst_hbm, out_hbm, *scr):
    dst_v = scr[0:NDST]
    dst_t = scr[NDST]
    src_all = scr[NDST + 1]
    rows_v = scr[NDST + 2]
    acc_sh = scr[NDST + 3]
    sem_g = scr[NDST + 4 : NDST + 4 + NBUF]
    sem_s = scr[NDST + 4 + NBUF : NDST + 4 + 2 * NBUF]
    sem_d = scr[NDST + 4 + 2 * NBUF :]

    c = lax.axis_index("c")
    s = lax.axis_index("s")

    base = pl.multiple_of((c * NS + s) * EDGES_PER_TILE, 8)
    out_c = out_hbm.at[c]

    pltpu.sync_copy(src_hbm.at[pl.ds(base, EDGES_PER_TILE)], src_all)

    def dst_load(i, db):
        off = pl.multiple_of(base + i * CF, 8)
        return pltpu.async_copy(dst_hbm.at[pl.ds(off, CF)], dst_v[db], sem_d[db])

    def gather(i, b):
        return pltpu.async_copy(
            g_hbm.at[src_all.at[pl.ds(i * CF, CF)]], rows_v.at[b], sem_g[b]
        )

    gat = [None] * NBUF
    sca = [None] * NBUF
    dld = [None] * NDST
    for j in range(3):
        dld[j] = dst_load(j, j)
    for j in range(2):
        gat[j] = gather(j, j)

    _zero_vmem(rows_v.at[2], ZROWS, D)
    for k in range(ROWS_PER_TILE // ZROWS):
        pltpu.sync_copy(
            rows_v.at[2, pl.ds(0, ZROWS)],
            acc_sh.at[pl.ds(s * ROWS_PER_TILE + k * ZROWS, ZROWS)],
        )
    plsc.subcore_barrier()
    for i in range(CHUNKS_F):
        b = i % NBUF
        db = i % NDST
        gat[b].wait()
        dld[db].wait()
        sca[b] = pltpu.async_copy(
            rows_v.at[b], acc_sh.at[dst_v[db]], sem_s[b], add=True
        )
        nxt = i + 2
        if nxt < CHUNKS_F:
            nb = nxt % NBUF
            if sca[nb] is not None:
                sca[nb].wait()
                sca[nb] = None
            gat[nb] = gather(nxt, nb)
        nd = i + 3
        if nd < CHUNKS_F:
            dld[nd % NDST] = dst_load(nd, nd % NDST)
    if sca[0] is not None:
        sca[0].wait()
        sca[0] = None
    tg = pltpu.async_copy(
        g_hbm.at[src_all.at[pl.ds(CHUNKS_F * CF, CTAIL)]],
        rows_v.at[0, pl.ds(0, CTAIL)],
        sem_g[0],
    )
    pltpu.sync_copy(
        dst_hbm.at[pl.ds(pl.multiple_of(base + CHUNKS_F * CF, 8), CTAIL)], dst_t
    )
    tg.wait()
    pltpu.sync_copy(rows_v.at[0, pl.ds(0, CTAIL)], acc_sh.at[dst_t], add=True)

    for b in range(NBUF):
        if sca[b] is not None:
            sca[b].wait()
    plsc.subcore_barrier()
    pltpu.sync_copy(
        acc_sh.at[pl.ds(s * ROWS_PER_TILE, ROWS_PER_TILE)],
        out_c.at[pl.ds(s * ROWS_PER_TILE, ROWS_PER_TILE)],
    )


_R = 2560
_GRID = N_PAD // _R


def _dinv_block(degp_ref):
    deg = jnp.sum(degp_ref[...], axis=0) + 1.0
    return lax.rsqrt(jnp.maximum(deg, 1.0))


def _tc_first_body(degp_ref, x_ref, w_ref, g_ref):
    dinv = _dinv_block(degp_ref)
    h = jnp.dot(x_ref[...], w_ref[...], preferred_element_type=jnp.float32)
    g_ref[...] = h * dinv[:, None]


def _tc_mid_body(sp_ref, g_ref, degp_ref, b_ref, w_ref, gn_ref):
    dinv = _dinv_block(degp_ref)
    pre = (sp_ref[0] + sp_ref[1] + g_ref[...]) * dinv[:, None] + b_ref[...]
    a = jnp.maximum(pre, 0.0)
    gn_ref[...] = (
        jnp.dot(a, w_ref[...], preferred_element_type=jnp.float32) * dinv[:, None]
    )


def _tc_last_body(sp_ref, g_ref, degp_ref, b_ref, out_ref):
    dinv = _dinv_block(degp_ref)
    out_ref[...] = (sp_ref[0] + sp_ref[1] + g_ref[...]) * dinv[:, None] + b_ref[...]


_degp_spec = pl.BlockSpec((NC * NS, _R), lambda i: (0, i))
_row_spec = pl.BlockSpec((_R, D), lambda i: (i, 0))
_sp_spec = pl.BlockSpec((NC, _R, D), lambda i: (0, i, 0))
_w_spec = pl.BlockSpec((D, D), lambda i: (0, 0))
_b_spec = pl.BlockSpec((1, D), lambda i: (0, 0))


def _tc_first(degp, x, w):
    return pl.pallas_call(
        _tc_first_body,
        grid=(_GRID,),
        in_specs=[_degp_spec, _row_spec, _w_spec],
        out_specs=_row_spec,
        out_shape=jax.ShapeDtypeStruct((N_PAD, D), jnp.float32),
    )(degp, x, w)


def _tc_mid(sp, g, degp, b, w):
    return pl.pallas_call(
        _tc_mid_body,
        grid=(_GRID,),
        in_specs=[_sp_spec, _row_spec, _degp_spec, _b_spec, _w_spec],
        out_specs=_row_spec,
        out_shape=jax.ShapeDtypeStruct((N_PAD, D), jnp.float32),
    )(sp, g, degp, b, w)


def _tc_last(sp, g, degp, b):
    return pl.pallas_call(
        _tc_last_body,
        grid=(_GRID,),
        in_specs=[_sp_spec, _row_spec, _degp_spec, _b_spec],
        out_specs=_row_spec,
        out_shape=jax.ShapeDtypeStruct((N_PAD, D), jnp.float32),
    )(sp, g, degp, b)


def kernel(x, edge_index, W1, b1, W2, b2, W3, b3):
    edge_index = edge_index.astype(jnp.int32)
    src = edge_index[0]
    dst = edge_index[1]
    b1 = b1.reshape(1, D)
    b2 = b2.reshape(1, D)
    b3 = b3.reshape(1, D)

    x_p = jnp.pad(x, ((0, N_PAD - N), (0, 0)))
    degp = _sc_degree(dst)
    g1 = _tc_first(degp, x_p, W1)
    s1 = _sc_scatter(g1, src, dst)
    g2 = _tc_mid(s1, g1, degp, b1, W2)
    s2 = _sc_scatter(g2, src, dst)
    g3 = _tc_mid(s2, g2, degp, b2, W3)
    s3 = _sc_scatter(g3, src, dst)
    return _tc_last(s3, g3, degp, b3)[:N]


# speedup vs baseline: 33.3801x; 1.0131x over previous
import functools

import jax
import jax.numpy as jnp
from jax import lax
from jax.experimental import pallas as pl
from jax.experimental.pallas import tpu as pltpu
from jax.experimental.pallas import tpu_sc as plsc

N = 10000
N_PAD = 10240
E = 320000
D = 128

NC = 2
NS = 16
EDGES_PER_TILE = E // (NC * NS)
ROWS_PER_TILE = N_PAD // NS
ZROWS = 80

CF = 120
CHUNKS_F = EDGES_PER_TILE // CF
CTAIL = EDGES_PER_TILE - CHUNKS_F * CF
NBUF = 3
NDST = 4

_MESH = plsc.VectorSubcoreMesh(core_axis_name="c", subcore_axis_name="s")


def _zero_vmem(ref, rows, cols):
    z = jnp.zeros((16,), jnp.float32)

    def body(i, _):
        for j in range(cols // 16):
            ref[i, pl.ds(j * 16, 16)] = z
        return 0

    lax.fori_loop(0, rows, body, 0)


@functools.partial(
    pl.kernel,
    mesh=_MESH,
    out_type=jax.ShapeDtypeStruct((NC * NS, N_PAD), jnp.float32),
    scratch_types=[
        pltpu.VMEM((EDGES_PER_TILE,), jnp.int32),
        pltpu.VMEM((N_PAD,), jnp.float32),
    ],
    compiler_params=pltpu.CompilerParams(needs_layout_passes=False),
)
def _sc_degree(dst_hbm, out_hbm, idx_all, deg_local):
    c = lax.axis_index("c")
    s = lax.axis_index("s")
    w = c * NS + s

    z = jnp.zeros((16,), jnp.float32)

    def zbody(i, _):
        deg_local[pl.ds(i * 16, 16)] = z
        return 0

    lax.fori_loop(0, N_PAD // 16, zbody, 0)

    base = pl.multiple_of(w * EDGES_PER_TILE, 8)
    pltpu.sync_copy(dst_hbm.at[pl.ds(base, EDGES_PER_TILE)], idx_all)

    one = jnp.ones((16,), jnp.float32)

    def body(k, _):
        idx_v = idx_all[pl.ds(k * 16, 16)]
        plsc.addupdate_scatter(deg_local, [idx_v], one)
        return 0

    lax.fori_loop(0, EDGES_PER_TILE // 16, body, 0)

    pltpu.sync_copy(deg_local, out_hbm.at[w])


@functools.partial(
    pl.kernel,
    mesh=_MESH,
    out_type=jax.ShapeDtypeStruct((NC, N_PAD, D), jnp.float32),
    scratch_types=(
        [pltpu.VMEM((CF,), jnp.int32) for _ in range(2 * NDST)]
        + [
            pltpu.VMEM((CTAIL,), jnp.int32),
            pltpu.VMEM((CTAIL,), jnp.int32),
            pltpu.VMEM((NBUF, CF, D), jnp.float32),
            pltpu.VMEM_SHARED((N_PAD, D), jnp.float32),
        ]
        + [pltpu.SemaphoreType.DMA for _ in range(2 * NBUF + 2 * NDST)]
    ),
)
def _sc_scatter(g_hbm, src_hbm, dst_hbm, out_hbm, *scr):
    src_v = scr[0:NDST]
    dst_v = scr[NDST : 2 * NDST]
    src_t = scr[2 * NDST]
    dst_t = scr[2 * NDST + 1]
    rows_v = scr[2 * NDST + 2]
    acc_sh = scr[2 * NDST + 3]
    k = 2 * NDST + 4
    sem_g = scr[k : k + NBUF]
    sem_s = scr[k + NBUF : k + 2 * NBUF]
    sem_i = scr[k + 2 * NBUF : k + 2 * NBUF + NDST]
    sem_d = scr[k + 2 * NBUF + NDST :]

    c = lax.axis_index("c")
    s = lax.axis_index("s")

    base = pl.multiple_of((c * NS + s) * EDGES_PER_TILE, 8)
    out_c = out_hbm.at[c]

    def idx_load(i, db):
        off = pl.multiple_of(base + i * CF, 8)
        return (
            pltpu.async_copy(src_hbm.at[pl.ds(off, CF)], src_v[db], sem_i[db]),
            pltpu.async_copy(dst_hbm.at[pl.ds(off, CF)], dst_v[db], sem_d[db]),
        )

    def gather(i, b):
        return pltpu.async_copy(g_hbm.at[src_v[i % NDST]], rows_v.at[b], sem_g[b])

    gat = [None] * NBUF
    sca = [None] * NBUF
    ild = [None] * NDST
    for j in range(3):
        ild[j] = idx_load(j, j)
    for j in range(2):
        ild[j][0].wait()
        gat[j] = gather(j, j)

    _zero_vmem(rows_v.at[2], ZROWS, D)
    for k in range(ROWS_PER_TILE // ZROWS):
        pltpu.sync_copy(
            rows_v.at[2, pl.ds(0, ZROWS)],
            acc_sh.at[pl.ds(s * ROWS_PER_TILE + k * ZROWS, ZROWS)],
        )
    plsc.subcore_barrier()
    for i in range(CHUNKS_F):
        b = i % NBUF
        db = i % NDST
        gat[b].wait()
        if ild[db] is not None and ild[db][1] is not None:
            ild[db][1].wait()
        sca[b] = pltpu.async_copy(
            rows_v.at[b], acc_sh.at[dst_v[db]], sem_s[b], add=True
        )
        nxt = i + 2
        if nxt < CHUNKS_F:
            nb = nxt % NBUF
            if sca[nb] is not None:
                sca[nb].wait()
                sca[nb] = None
            if ild[nxt % NDST] is not None and ild[nxt % NDST][0] is not None:
                ild[nxt % NDST][0].wait()
                ild[nxt % NDST] = (None, ild[nxt % NDST][1])
            gat[nb] = gather(nxt, nb)
        nd = i + 3
        if nd < CHUNKS_F:
            ild[nd % NDST] = idx_load(nd, nd % NDST)
    if sca[0] is not None:
        sca[0].wait()
        sca[0] = None
    toff = pl.multiple_of(base + CHUNKS_F * CF, 8)
    tsem = pltpu.async_copy(src_hbm.at[pl.ds(toff, CTAIL)], src_t, sem_i[0])
    pltpu.sync_copy(dst_hbm.at[pl.ds(toff, CTAIL)], dst_t)
    tsem.wait()
    tg = pltpu.async_copy(
        g_hbm.at[src_t], rows_v.at[0, pl.ds(0, CTAIL)], sem_g[0]
    )
    tg.wait()
    pltpu.sync_copy(rows_v.at[0, pl.ds(0, CTAIL)], acc_sh.at[dst_t], add=True)

    for b in range(NBUF):
        if sca[b] is not None:
            sca[b].wait()
    plsc.subcore_barrier()
    pltpu.sync_copy(
        acc_sh.at[pl.ds(s * ROWS_PER_TILE, ROWS_PER_TILE)],
        out_c.at[pl.ds(s * ROWS_PER_TILE, ROWS_PER_TILE)],
    )


_R = 2560
_GRID = N_PAD // _R


def _dinv_block(degp_ref):
    deg = jnp.sum(degp_ref[...], axis=0) + 1.0
    return lax.rsqrt(jnp.maximum(deg, 1.0))


def _tc_first_body(degp_ref, x_ref, w_ref, g_ref):
    dinv = _dinv_block(degp_ref)
    h = jnp.dot(x_ref[...], w_ref[...], preferred_element_type=jnp.float32)
    g_ref[...] = h * dinv[:, None]


def _tc_mid_body(sp_ref, g_ref, degp_ref, b_ref, w_ref, gn_ref):
    dinv = _dinv_block(degp_ref)
    pre = (sp_ref[0] + sp_ref[1] + g_ref[...]) * dinv[:, None] + b_ref[...]
    a = jnp.maximum(pre, 0.0)
    gn_ref[...] = (
        jnp.dot(a, w_ref[...], preferred_element_type=jnp.float32) * dinv[:, None]
    )


def _tc_last_body(sp_ref, g_ref, degp_ref, b_ref, out_ref):
    dinv = _dinv_block(degp_ref)
    out_ref[...] = (sp_ref[0] + sp_ref[1] + g_ref[...]) * dinv[:, None] + b_ref[...]


_degp_spec = pl.BlockSpec((NC * NS, _R), lambda i: (0, i))
_row_spec = pl.BlockSpec((_R, D), lambda i: (i, 0))
_sp_spec = pl.BlockSpec((NC, _R, D), lambda i: (0, i, 0))
_w_spec = pl.BlockSpec((D, D), lambda i: (0, 0))
_b_spec = pl.BlockSpec((1, D), lambda i: (0, 0))


def _tc_first(degp, x, w):
    return pl.pallas_call(
        _tc_first_body,
        grid=(_GRID,),
        in_specs=[_degp_spec, _row_spec, _w_spec],
        out_specs=_row_spec,
        out_shape=jax.ShapeDtypeStruct((N_PAD, D), jnp.float32),
    )(degp, x, w)


def _tc_mid(sp, g, degp, b, w):
    return pl.pallas_call(
        _tc_mid_body,
        grid=(_GRID,),
        in_specs=[_sp_spec, _row_spec, _degp_spec, _b_spec, _w_spec],
        out_specs=_row_spec,
        out_shape=jax.ShapeDtypeStruct((N_PAD, D), jnp.float32),
    )(sp, g, degp, b, w)


def _tc_last(sp, g, degp, b):
    return pl.pallas_call(
        _tc_last_body,
        grid=(_GRID,),
        in_specs=[_sp_spec, _row_spec, _degp_spec, _b_spec],
        out_specs=_row_spec,
        out_shape=jax.ShapeDtypeStruct((N_PAD, D), jnp.float32),
    )(sp, g, degp, b)


def kernel(x, edge_index, W1, b1, W2, b2, W3, b3):
    edge_index = edge_index.astype(jnp.int32)
    src = edge_index[0]
    dst = edge_index[1]
    b1 = b1.reshape(1, D)
    b2 = b2.reshape(1, D)
    b3 = b3.reshape(1, D)

    x_p = jnp.pad(x, ((0, N_PAD - N), (0, 0)))
    degp = _sc_degree(dst)
    g1 = _tc_first(degp, x_p, W1)
    s1 = _sc_scatter(g1, src, dst)
    g2 = _tc_mid(s1, g1, degp, b1, W2)
    s2 = _sc_scatter(g2, src, dst)
    g3 = _tc_mid(s2, g2, degp, b2, W3)
    s3 = _sc_scatter(g3, src, dst)
    return _tc_last(s3, g3, degp, b3)[:N]
